# Initial kernel scaffold; baseline (speedup 1.0000x reference)
import jax
import jax.numpy as jnp
from jax.experimental import pallas as pl


def kernel(params, node_features, edge_dist, chemical_x, quantum_x, edge_index, batch, quantum_mask):
    raise NotImplementedError("write your pallas kernel here")


# speedup vs baseline: 2.2038x; 2.2038x over previous
import functools

import jax
import jax.numpy as jnp
from jax import lax
from jax.experimental import pallas as pl
from jax.experimental.pallas import tpu as pltpu
from jax.experimental.pallas import tpu_sc as plsc

N_NODES = 10000
N_EDGES = 320000
B = 128
HID = 128
RBF = 50

NC = 2
NS = 16
NW = NC * NS
EPW = N_EDGES // NW
CH = 80
NCH = EPW // CH
NPAD = 10240
RPS = NPAD // NS


def _silu(x):
    return x * jax.nn.sigmoid(x)


def _gelu(x):
    return 0.5 * x * (1.0 + lax.erf(x * 0.7071067811865476))


def _ln(x, g, b):
    m = jnp.mean(x, axis=-1, keepdims=True)
    v = jnp.mean((x - m) ** 2, axis=-1, keepdims=True)
    return (x - m) / jnp.sqrt(v + 1e-5) * g + b


def _embed_body(nf_ref, w_ref, b_ref, o_ref):
    o_ref[...] = _silu(jnp.dot(nf_ref[...], w_ref[...],
                               preferred_element_type=jnp.float32) + b_ref[...])


def _node_embed(nf8, w8, b):
    grid = (5,)
    return pl.pallas_call(
        _embed_body,
        grid=grid,
        in_specs=[
            pl.BlockSpec((2000, 8), lambda i: (i, 0)),
            pl.BlockSpec((8, HID), lambda i: (0, 0)),
            pl.BlockSpec((1, HID), lambda i: (0, 0)),
        ],
        out_specs=pl.BlockSpec((2000, HID), lambda i: (i, 0)),
        out_shape=jax.ShapeDtypeStruct((N_NODES, HID), jnp.float32),
    )(nf8, w8, b)


_EBLK = 2000
_NEB = N_EDGES // _EBLK


def _filters_body(ed_ref, w1_ref, b1_ref, w2_ref, b2_ref, o_ref):
    d = ed_ref[0, 0].reshape(_EBLK, 1)
    centers = lax.broadcasted_iota(jnp.int32, (_EBLK, RBF), 1).astype(jnp.float32) * (6.0 / (RBF - 1))
    rbf = jnp.exp(-10.0 * (d - centers) ** 2)
    t = _silu(jnp.dot(rbf, w1_ref[0], preferred_element_type=jnp.float32) + b1_ref[0])
    o_ref[0, 0] = jnp.dot(t, w2_ref[0], preferred_element_type=jnp.float32) + b2_ref[0]


def _edge_filters(ed, w1s, b1s, w2s, b2s):
    grid = (3, _NEB)
    return pl.pallas_call(
        _filters_body,
        grid=grid,
        in_specs=[
            pl.BlockSpec((1, 1, _EBLK), lambda l, j: (j, 0, 0)),
            pl.BlockSpec((1, RBF, HID), lambda l, j: (l, 0, 0)),
            pl.BlockSpec((1, 1, HID), lambda l, j: (l, 0, 0)),
            pl.BlockSpec((1, HID, HID), lambda l, j: (l, 0, 0)),
            pl.BlockSpec((1, 1, HID), lambda l, j: (l, 0, 0)),
        ],
        out_specs=pl.BlockSpec((1, 1, _EBLK, HID), lambda l, j: (l, j, 0, 0)),
        out_shape=jax.ShapeDtypeStruct((3, _NEB, _EBLK, HID), jnp.float32),
    )(ed.reshape(_NEB, 1, _EBLK), w1s.reshape(3, RBF, HID),
      b1s.reshape(3, 1, HID), w2s.reshape(3, HID, HID), b2s.reshape(3, 1, HID))


def _sc_body(h_hbm, src_hbm, dst_hbm, w_hbm, zeros_hbm, out_hbm,
             idx_v, dstv_v, rows_v, w_v, acc_sh, sem):
    cid = lax.axis_index("c")
    sid = lax.axis_index("s")
    wid = sid * NC + cid

    pltpu.sync_copy(zeros_hbm, acc_sh.at[pl.ds(sid * RPS, RPS)])
    plsc.subcore_barrier()

    base = wid * EPW

    def chunk(j, carry):
        off = base + j * CH
        pltpu.sync_copy(src_hbm.at[pl.ds(off, CH)], idx_v)
        pltpu.sync_copy(dst_hbm.at[pl.ds(off, CH)], dstv_v)
        pltpu.async_copy(h_hbm.at[idx_v], rows_v, sem).wait()
        pltpu.sync_copy(w_hbm.at[pl.ds(off, CH)], w_v)

        def mul_row(e, c2):
            for cc in range(HID // 16):
                s = pl.ds(cc * 16, 16)
                rows_v[e, s] = rows_v[e, s] * w_v[e, s]
            return c2

        lax.fori_loop(0, CH, mul_row, 0)
        pltpu.sync_copy(rows_v, acc_sh.at[dstv_v], add=True)
        return carry

    lax.fori_loop(0, NCH, chunk, 0)
    plsc.subcore_barrier()
    pltpu.sync_copy(acc_sh.at[pl.ds(sid * RPS, RPS)],
                    out_hbm.at[cid, pl.ds(sid * RPS, RPS)])


@functools.cache
def _make_sc_agg():
    return pl.kernel(
        _sc_body,
        mesh=plsc.VectorSubcoreMesh(core_axis_name="c", subcore_axis_name="s"),
        out_type=jax.ShapeDtypeStruct((NC, NPAD, HID), jnp.float32),
        scratch_types=[
            pltpu.VMEM((CH,), jnp.int32),
            pltpu.VMEM((CH,), jnp.int32),
            pltpu.VMEM((CH, HID), jnp.float32),
            pltpu.VMEM((CH, HID), jnp.float32),
            pltpu.VMEM_SHARED((NPAD, HID), jnp.float32),
            pltpu.SemaphoreType.DMA,
        ],
    )


def _sc_agg(h, src, dst, w, zeros):
    return _make_sc_agg()(h, src, dst, w, zeros)


def _hup_body(p_ref, h_ref, w_ref, b_ref, o_ref):
    agg = p_ref[0] + p_ref[1]
    o_ref[...] = h_ref[...] + _silu(
        jnp.dot(agg, w_ref[...], preferred_element_type=jnp.float32) + b_ref[...])


def _h_update(partials, h, lw, lb):
    grid = (5,)
    return pl.pallas_call(
        _hup_body,
        grid=grid,
        in_specs=[
            pl.BlockSpec((NC, 2000, HID), lambda i: (0, i, 0)),

            pl.BlockSpec((2000, HID), lambda i: (i, 0)),
            pl.BlockSpec((HID, HID), lambda i: (0, 0)),
            pl.BlockSpec((1, HID), lambda i: (0, 0)),
        ],
        out_specs=pl.BlockSpec((2000, HID), lambda i: (i, 0)),
        out_shape=jax.ShapeDtypeStruct((N_NODES, HID), jnp.float32),
    )(partials, h, lw, lb)


def _head_body(h_ref, batch_ref, chem_ref, qx_ref, qmask_ref,
               gw1, gb1, gw2r, gb2, pw, pb, pg, pbeta,
               cw1, cb1, cg1, cbeta1, cw2, cb2, cg2, cbeta2,
               qw1, qb1, qw2, qb2, qmiss,
               fvw, fvb, fow, fob, fg, fbeta,
               hw1a, hw1b, hb1, hg1, hbeta1, hw2, hb2, hw3r, hb3,
               o_ref):
    h = h_ref[...]
    gate_h = _silu(jnp.dot(h, gw1[...], preferred_element_type=jnp.float32) + gb1[...])
    gate = jnp.sum(gate_h * gw2r[...], axis=1, keepdims=True) + gb2[...]

    cols = lax.broadcasted_iota(jnp.int32, (N_NODES, B), 1)
    maskf = (batch_ref[...] == cols).astype(jnp.float32)

    gmax_g = jnp.max(jnp.where(maskf > 0.0, gate, -1e30), axis=0, keepdims=True)
    gmax_n = lax.dot_general(maskf, gmax_g, (((1,), (1,)), ((), ())),
                             preferred_element_type=jnp.float32)
    gexp = jnp.exp(gate - gmax_n)
    gsum_g = lax.dot_general(maskf, gexp, (((0,), (0,)), ((), ())),
                             preferred_element_type=jnp.float32)
    gsum_n = lax.dot_general(maskf, gsum_g, (((1,), (0,)), ((), ())),
                             preferred_element_type=jnp.float32) + 1e-8
    alpha = gexp / gsum_n
    hg = lax.dot_general(maskf, alpha * h, (((0,), (0,)), ((), ())),
                         preferred_element_type=jnp.float32)

    g = _gelu(_ln(jnp.dot(hg, pw[...], preferred_element_type=jnp.float32) + pb[...],
                  pg[...], pbeta[...]))

    c = _gelu(_ln(jnp.dot(chem_ref[...], cw1[...], preferred_element_type=jnp.float32)
                  + cb1[...], cg1[...], cbeta1[...]))
    c = _gelu(_ln(jnp.dot(c, cw2[...], preferred_element_type=jnp.float32) + cb2[...],
                  cg2[...], cbeta2[...]))

    qf = _gelu(jnp.dot(qx_ref[...], qw1[...], preferred_element_type=jnp.float32) + qb1[...])
    qf = _gelu(jnp.dot(qf, qw2[...], preferred_element_type=jnp.float32) + qb2[...])
    q_out = jnp.where(qmask_ref[...] > 0, qf, qmiss[...])

    vv = jnp.dot(g, fvw[...], preferred_element_type=jnp.float32) + fvb[...]
    fo = jnp.dot(vv, fow[...], preferred_element_type=jnp.float32) + fob[...]
    fo = _ln(fo + c, fg[...], fbeta[...])

    x = _gelu(_ln(jnp.dot(fo, hw1a[...], preferred_element_type=jnp.float32)
                  + jnp.dot(q_out, hw1b[...], preferred_element_type=jnp.float32)
                  + hb1[...], hg1[...], hbeta1[...]))
    x = _gelu(jnp.dot(x, hw2[...], preferred_element_type=jnp.float32) + hb2[...])
    o_ref[...] = jnp.sum(x * hw3r[...], axis=1, keepdims=True) + hb3[...]


def _head(h, batch2d, chem, qx, qmask2d, weights):
    return pl.pallas_call(
        _head_body,
        out_shape=jax.ShapeDtypeStruct((B, 1), jnp.float32),
    )(h, batch2d, chem, qx, qmask2d, *weights)


def kernel(params, node_features, edge_dist, chemical_x, quantum_x, edge_index,
           batch, quantum_mask):
    p = params
    f32 = jnp.float32

    nf8 = jnp.pad(node_features, ((0, 0), (0, 5)))
    w8 = jnp.pad(p['ne_W'], ((0, 5), (0, 0)))
    h = _node_embed(nf8, w8, p['ne_b'].reshape(1, HID))

    w1s = jnp.stack([p['int%d' % i]['fW1'] for i in range(3)])
    b1s = jnp.stack([p['int%d' % i]['fb1'] for i in range(3)])
    w2s = jnp.stack([p['int%d' % i]['fW2'] for i in range(3)])
    b2s = jnp.stack([p['int%d' % i]['fb2'] for i in range(3)])
    W = _edge_filters(edge_dist, w1s, b1s, w2s, b2s).reshape(3, N_EDGES, HID)

    src = edge_index[0].astype(jnp.int32)
    dst = edge_index[1].astype(jnp.int32)
    zeros = jnp.zeros((RPS, HID), f32)

    for l in range(3):
        partials = _sc_agg(h, src, dst, W[l], zeros)
        h = _h_update(partials, h, p['int%d' % l]['lW'],
                      p['int%d' % l]['lb'].reshape(1, HID))

    weights = [
        p['gate_W1'], p['gate_b1'].reshape(1, 64),
        p['gate_W2'].reshape(1, 64), p['gate_b2'].reshape(1, 1),
        p['proj_W'], p['proj_b'].reshape(1, HID),
        p['proj_g'].reshape(1, HID), p['proj_beta'].reshape(1, HID),
        p['chem_W1'], p['chem_b1'].reshape(1, 256),
        p['chem_g1'].reshape(1, 256), p['chem_beta1'].reshape(1, 256),
        p['chem_W2'], p['chem_b2'].reshape(1, 128),
        p['chem_g2'].reshape(1, 128), p['chem_beta2'].reshape(1, 128),
        p['qm_W1'], p['qm_b1'].reshape(1, 64),
        p['qm_W2'], p['qm_b2'].reshape(1, 64), p['qm_missing'].reshape(1, 64),
        p['fu_vW'], p['fu_vb'].reshape(1, 128),
        p['fu_oW'], p['fu_ob'].reshape(1, 128),
        p['fu_g'].reshape(1, 128), p['fu_beta'].reshape(1, 128),
        p['hd_W1'][:128], p['hd_W1'][128:],
        p['hd_b1'].reshape(1, 256),
        p['hd_g1'].reshape(1, 256), p['hd_beta1'].reshape(1, 256),
        p['hd_W2'], p['hd_b2'].reshape(1, 128),
        p['hd_W3'].reshape(1, 128), p['hd_b3'].reshape(1, 1),
    ]
    preds = _head(h, batch.reshape(N_NODES, 1).astype(jnp.int32),
                  chemical_x, quantum_x,
                  quantum_mask.reshape(B, 1).astype(jnp.int32), weights)
    return preds.reshape(B)


# speedup vs baseline: 3.8197x; 1.7333x over previous
import functools

import jax
import jax.numpy as jnp
from jax import lax
from jax.experimental import pallas as pl
from jax.experimental.pallas import tpu as pltpu
from jax.experimental.pallas import tpu_sc as plsc

N_NODES = 10000
N_EDGES = 320000
B = 128
HID = 128
RBF = 50

NC = 2
NS = 16
NW = NC * NS
EPW = N_EDGES // NW
CH = 80
NCH = EPW // CH
NPAD = 10240
RPS = NPAD // NS


def _silu(x):
    return x * jax.nn.sigmoid(x)


def _gelu(x):
    return 0.5 * x * (1.0 + lax.erf(x * 0.7071067811865476))


def _ln(x, g, b):
    m = jnp.mean(x, axis=-1, keepdims=True)
    v = jnp.mean((x - m) ** 2, axis=-1, keepdims=True)
    return (x - m) / jnp.sqrt(v + 1e-5) * g + b


def _embed_body(nf_ref, w_ref, b_ref, o_ref):
    o_ref[...] = _silu(jnp.dot(nf_ref[...], w_ref[...],
                               preferred_element_type=jnp.float32) + b_ref[...])


def _node_embed(nf8, w8, b):
    grid = (5,)
    return pl.pallas_call(
        _embed_body,
        grid=grid,
        in_specs=[
            pl.BlockSpec((2000, 8), lambda i: (i, 0)),
            pl.BlockSpec((8, HID), lambda i: (0, 0)),
            pl.BlockSpec((1, HID), lambda i: (0, 0)),
        ],
        out_specs=pl.BlockSpec((2000, HID), lambda i: (i, 0)),
        out_shape=jax.ShapeDtypeStruct((N_NODES, HID), jnp.float32),
    )(nf8, w8, b)


_EBLK = 2000
_NEB = N_EDGES // _EBLK


def _filters_body(ed_ref, w1_ref, b1_ref, w2_ref, b2_ref, o_ref):
    d = ed_ref[0, 0].reshape(_EBLK, 1)
    centers = lax.broadcasted_iota(jnp.int32, (_EBLK, RBF), 1).astype(jnp.float32) * (6.0 / (RBF - 1))
    rbf = jnp.exp(-10.0 * (d - centers) ** 2)
    t = _silu(jnp.dot(rbf, w1_ref[0], preferred_element_type=jnp.float32) + b1_ref[0])
    o_ref[0, 0] = (jnp.dot(t, w2_ref[0], preferred_element_type=jnp.float32)
                   + b2_ref[0])


def _edge_filters(ed, w1s, b1s, w2s, b2s):
    grid = (3, _NEB)
    return pl.pallas_call(
        _filters_body,
        grid=grid,
        in_specs=[
            pl.BlockSpec((1, 1, _EBLK), lambda l, j: (j, 0, 0)),
            pl.BlockSpec((1, RBF, HID), lambda l, j: (l, 0, 0)),
            pl.BlockSpec((1, 1, HID), lambda l, j: (l, 0, 0)),
            pl.BlockSpec((1, HID, HID), lambda l, j: (l, 0, 0)),
            pl.BlockSpec((1, 1, HID), lambda l, j: (l, 0, 0)),
        ],
        out_specs=pl.BlockSpec((1, 1, _EBLK, HID), lambda l, j: (l, j, 0, 0)),
        out_shape=jax.ShapeDtypeStruct((3, _NEB, _EBLK, HID), jnp.float32),
    )(ed.reshape(_NEB, 1, _EBLK), w1s.reshape(3, RBF, HID),
      b1s.reshape(3, 1, HID), w2s.reshape(3, HID, HID), b2s.reshape(3, 1, HID))


def _sc_body(h_hbm, src_hbm, dst_hbm, w_hbm, zeros_hbm, out_hbm,
             idx0, idx1, rows0, rows1, w0, w1, dst0, dst1,
             acc_sh, si0, si1, sg0, sg1, sw0, sw1, sd0, sd1):
    cid = lax.axis_index("c")
    sid = lax.axis_index("s")
    wid = sid * NC + cid

    pltpu.sync_copy(zeros_hbm, acc_sh.at[pl.ds(sid * RPS, RPS)])
    plsc.subcore_barrier()
    base = wid * EPW

    def fire_idx(j, idxb, si):
        pltpu.async_copy(src_hbm.at[pl.ds(base + j * CH, CH)], idxb, si)

    def wait_idx(j, idxb, si):
        pltpu.make_async_copy(src_hbm.at[pl.ds(base + j * CH, CH)], idxb, si).wait()

    def fire_main(j, idxb, rows, w, dstb, sg, sw, sd):
        pltpu.async_copy(h_hbm.at[idxb], rows, sg)
        pltpu.async_copy(w_hbm.at[pl.ds(base + j * CH, CH)], w, sw)
        pltpu.async_copy(dst_hbm.at[pl.ds(base + j * CH, CH)], dstb, sd)

    def process(j, idxb, rows, w, dstb, sg, sw, sd):
        pltpu.make_async_copy(h_hbm.at[idxb], rows, sg).wait()
        pltpu.make_async_copy(w_hbm.at[pl.ds(base + j * CH, CH)], w, sw).wait()
        pltpu.make_async_copy(dst_hbm.at[pl.ds(base + j * CH, CH)], dstb, sd).wait()

        def rowblk(r, c2):
            for rr in range(8):
                e = r * 8 + rr
                for cc in range(HID // 16):
                    s = pl.ds(cc * 16, 16)
                    rows[e, s] = rows[e, s] * w[e, s]
            return c2

        lax.fori_loop(0, CH // 8, rowblk, 0)
        pltpu.sync_copy(rows, acc_sh.at[dstb], add=True)

    b0 = (idx0, rows0, w0, dst0, sg0, sw0, sd0)
    b1 = (idx1, rows1, w1, dst1, sg1, sw1, sd1)

    fire_idx(0, idx0, si0)
    fire_idx(1, idx1, si1)
    wait_idx(0, idx0, si0)
    fire_main(0, *b0)

    def pair(i, carry):
        j0 = i * 2
        j1 = j0 + 1
        wait_idx(j1, idx1, si1)
        fire_main(j1, *b1)

        @pl.when(j0 + 2 < NCH)
        def _():
            fire_idx(j0 + 2, idx0, si0)

        process(j0, *b0)

        @pl.when(j0 + 2 < NCH)
        def _():
            wait_idx(j0 + 2, idx0, si0)
            fire_main(j0 + 2, *b0)

        @pl.when(j1 + 2 < NCH)
        def _():
            fire_idx(j1 + 2, idx1, si1)

        process(j1, *b1)
        return carry

    lax.fori_loop(0, NCH // 2, pair, 0)
    process(NCH - 1, *b0)

    plsc.subcore_barrier()
    pltpu.sync_copy(acc_sh.at[pl.ds(sid * RPS, RPS)],
                    out_hbm.at[cid, pl.ds(sid * RPS, RPS)])


@functools.cache
def _make_sc_agg():
    return pl.kernel(
        _sc_body,
        mesh=plsc.VectorSubcoreMesh(core_axis_name="c", subcore_axis_name="s"),
        out_type=jax.ShapeDtypeStruct((NC, NPAD, HID), jnp.float32),
        scratch_types=[
            pltpu.VMEM((CH,), jnp.int32),
            pltpu.VMEM((CH,), jnp.int32),
            pltpu.VMEM((CH, HID), jnp.float32),
            pltpu.VMEM((CH, HID), jnp.float32),
            pltpu.VMEM((CH, HID), jnp.float32),
            pltpu.VMEM((CH, HID), jnp.float32),
            pltpu.VMEM((CH,), jnp.int32),
            pltpu.VMEM((CH,), jnp.int32),
            pltpu.VMEM_SHARED((NPAD, HID), jnp.float32),
        ] + [pltpu.SemaphoreType.DMA] * 8,
    )


def _sc_agg(h, src, dst, w, zeros):
    return _make_sc_agg()(h, src, dst, w, zeros)


def _hup_body(p_ref, h_ref, w_ref, b_ref, o_ref):
    agg = p_ref[0] + p_ref[1]
    o_ref[...] = h_ref[...] + _silu(
        jnp.dot(agg, w_ref[...], preferred_element_type=jnp.float32) + b_ref[...])


def _h_update(partials, h, lw, lb):
    grid = (5,)
    return pl.pallas_call(
        _hup_body,
        grid=grid,
        in_specs=[
            pl.BlockSpec((NC, 2000, HID), lambda i: (0, i, 0)),

            pl.BlockSpec((2000, HID), lambda i: (i, 0)),
            pl.BlockSpec((HID, HID), lambda i: (0, 0)),
            pl.BlockSpec((1, HID), lambda i: (0, 0)),
        ],
        out_specs=pl.BlockSpec((2000, HID), lambda i: (i, 0)),
        out_shape=jax.ShapeDtypeStruct((N_NODES, HID), jnp.float32),
    )(partials, h, lw, lb)


def _head_body(h_ref, batch_ref, chem_ref, qx_ref, qmask_ref,
               gw1, gb1, gw2r, gb2, pw, pb, pg, pbeta,
               cw1, cb1, cg1, cbeta1, cw2, cb2, cg2, cbeta2,
               qw1, qb1, qw2, qb2, qmiss,
               fvw, fvb, fow, fob, fg, fbeta,
               hw1a, hw1b, hb1, hg1, hbeta1, hw2, hb2, hw3r, hb3,
               o_ref):
    h = h_ref[...]
    gate_h = _silu(jnp.dot(h, gw1[...], preferred_element_type=jnp.float32) + gb1[...])
    gate = jnp.sum(gate_h * gw2r[...], axis=1, keepdims=True) + gb2[...]

    cols = lax.broadcasted_iota(jnp.int32, (N_NODES, B), 1)
    maskf = (batch_ref[...] == cols).astype(jnp.float32)

    gmax_g = jnp.max(jnp.where(maskf > 0.0, gate, -1e30), axis=0, keepdims=True)
    gmax_n = lax.dot_general(maskf, gmax_g, (((1,), (1,)), ((), ())),
                             preferred_element_type=jnp.float32)
    gexp = jnp.exp(gate - gmax_n)
    gsum_g = lax.dot_general(maskf, gexp, (((0,), (0,)), ((), ())),
                             preferred_element_type=jnp.float32)
    gsum_n = lax.dot_general(maskf, gsum_g, (((1,), (0,)), ((), ())),
                             preferred_element_type=jnp.float32) + 1e-8
    alpha = gexp / gsum_n
    hg = lax.dot_general(maskf, alpha * h, (((0,), (0,)), ((), ())),
                         preferred_element_type=jnp.float32)

    g = _gelu(_ln(jnp.dot(hg, pw[...], preferred_element_type=jnp.float32) + pb[...],
                  pg[...], pbeta[...]))

    c = _gelu(_ln(jnp.dot(chem_ref[...], cw1[...], preferred_element_type=jnp.float32)
                  + cb1[...], cg1[...], cbeta1[...]))
    c = _gelu(_ln(jnp.dot(c, cw2[...], preferred_element_type=jnp.float32) + cb2[...],
                  cg2[...], cbeta2[...]))

    qf = _gelu(jnp.dot(qx_ref[...], qw1[...], preferred_element_type=jnp.float32) + qb1[...])
    qf = _gelu(jnp.dot(qf, qw2[...], preferred_element_type=jnp.float32) + qb2[...])
    q_out = jnp.where(qmask_ref[...] > 0, qf, qmiss[...])

    vv = jnp.dot(g, fvw[...], preferred_element_type=jnp.float32) + fvb[...]
    fo = jnp.dot(vv, fow[...], preferred_element_type=jnp.float32) + fob[...]
    fo = _ln(fo + c, fg[...], fbeta[...])

    x = _gelu(_ln(jnp.dot(fo, hw1a[...], preferred_element_type=jnp.float32)
                  + jnp.dot(q_out, hw1b[...], preferred_element_type=jnp.float32)
                  + hb1[...], hg1[...], hbeta1[...]))
    x = _gelu(jnp.dot(x, hw2[...], preferred_element_type=jnp.float32) + hb2[...])
    o_ref[...] = jnp.sum(x * hw3r[...], axis=1, keepdims=True) + hb3[...]


def _head(h, batch2d, chem, qx, qmask2d, weights):
    return pl.pallas_call(
        _head_body,
        out_shape=jax.ShapeDtypeStruct((B, 1), jnp.float32),
    )(h, batch2d, chem, qx, qmask2d, *weights)


def kernel(params, node_features, edge_dist, chemical_x, quantum_x, edge_index,
           batch, quantum_mask):
    p = params
    f32 = jnp.float32

    nf8 = jnp.pad(node_features, ((0, 0), (0, 5)))
    w8 = jnp.pad(p['ne_W'], ((0, 5), (0, 0)))
    h = _node_embed(nf8, w8, p['ne_b'].reshape(1, HID))

    w1s = jnp.stack([p['int%d' % i]['fW1'] for i in range(3)])
    b1s = jnp.stack([p['int%d' % i]['fb1'] for i in range(3)])
    w2s = jnp.stack([p['int%d' % i]['fW2'] for i in range(3)])
    b2s = jnp.stack([p['int%d' % i]['fb2'] for i in range(3)])
    W = _edge_filters(edge_dist, w1s, b1s, w2s, b2s).reshape(3, N_EDGES, HID)

    src = edge_index[0].astype(jnp.int32)
    dst = edge_index[1].astype(jnp.int32)
    zeros = jnp.zeros((RPS, HID), jnp.float32)

    for l in range(3):
        partials = _sc_agg(h, src, dst, W[l], zeros)
        h = _h_update(partials, h, p['int%d' % l]['lW'],
                      p['int%d' % l]['lb'].reshape(1, HID))

    weights = [
        p['gate_W1'], p['gate_b1'].reshape(1, 64),
        p['gate_W2'].reshape(1, 64), p['gate_b2'].reshape(1, 1),
        p['proj_W'], p['proj_b'].reshape(1, HID),
        p['proj_g'].reshape(1, HID), p['proj_beta'].reshape(1, HID),
        p['chem_W1'], p['chem_b1'].reshape(1, 256),
        p['chem_g1'].reshape(1, 256), p['chem_beta1'].reshape(1, 256),
        p['chem_W2'], p['chem_b2'].reshape(1, 128),
        p['chem_g2'].reshape(1, 128), p['chem_beta2'].reshape(1, 128),
        p['qm_W1'], p['qm_b1'].reshape(1, 64),
        p['qm_W2'], p['qm_b2'].reshape(1, 64), p['qm_missing'].reshape(1, 64),
        p['fu_vW'], p['fu_vb'].reshape(1, 128),
        p['fu_oW'], p['fu_ob'].reshape(1, 128),
        p['fu_g'].reshape(1, 128), p['fu_beta'].reshape(1, 128),
        p['hd_W1'][:128], p['hd_W1'][128:],
        p['hd_b1'].reshape(1, 256),
        p['hd_g1'].reshape(1, 256), p['hd_beta1'].reshape(1, 256),
        p['hd_W2'], p['hd_b2'].reshape(1, 128),
        p['hd_W3'].reshape(1, 128), p['hd_b3'].reshape(1, 1),
    ]
    preds = _head(h, batch.reshape(N_NODES, 1).astype(jnp.int32),
                  chemical_x, quantum_x,
                  quantum_mask.reshape(B, 1).astype(jnp.int32), weights)
    return preds.reshape(B)


# speedup vs baseline: 3.8400x; 1.0053x over previous
import functools

import jax
import jax.numpy as jnp
from jax import lax
from jax.experimental import pallas as pl
from jax.experimental.pallas import tpu as pltpu
from jax.experimental.pallas import tpu_sc as plsc

N_NODES = 10000
N_EDGES = 320000
B = 128
HID = 128
RBF = 50

NC = 2
NS = 16
NW = NC * NS
EPW = N_EDGES // NW
CH = 80
NCH = EPW // CH
NPAD = 10240
RPS = NPAD // NS


def _silu(x):
    return x * jax.nn.sigmoid(x)


def _gelu(x):
    return 0.5 * x * (1.0 + lax.erf(x * 0.7071067811865476))


def _ln(x, g, b):
    m = jnp.mean(x, axis=-1, keepdims=True)
    v = jnp.mean((x - m) ** 2, axis=-1, keepdims=True)
    return (x - m) / jnp.sqrt(v + 1e-5) * g + b


def _pack2(x):
    lo = pltpu.bitcast(x[:, :64].astype(jnp.bfloat16).astype(jnp.float32), jnp.uint32)
    hi = pltpu.bitcast(x[:, 64:].astype(jnp.bfloat16).astype(jnp.float32), jnp.uint32)
    return pltpu.bitcast((lo >> 16) | hi, jnp.float32)


def _embed_body(nf_ref, w_ref, b_ref, o_ref):
    o_ref[...] = _silu(jnp.dot(nf_ref[...], w_ref[...],
                               preferred_element_type=jnp.float32) + b_ref[...])


def _node_embed(nf8, w8, b):
    grid = (5,)
    return pl.pallas_call(
        _embed_body,
        grid=grid,
        in_specs=[
            pl.BlockSpec((2000, 8), lambda i: (i, 0)),
            pl.BlockSpec((8, HID), lambda i: (0, 0)),
            pl.BlockSpec((1, HID), lambda i: (0, 0)),
        ],
        out_specs=pl.BlockSpec((2000, HID), lambda i: (i, 0)),
        out_shape=jax.ShapeDtypeStruct((N_NODES, HID), jnp.float32),
    )(nf8, w8, b)


_EBLK = 2000
_NEB = N_EDGES // _EBLK


def _filters_body(ed_ref, w1_ref, b1_ref, w2_ref, b2_ref, o_ref):
    d = ed_ref[0, 0].reshape(_EBLK, 1)
    centers = lax.broadcasted_iota(jnp.int32, (_EBLK, RBF), 1).astype(jnp.float32) * (6.0 / (RBF - 1))
    rbf = jnp.exp(-10.0 * (d - centers) ** 2)
    t = _silu(jnp.dot(rbf, w1_ref[0], preferred_element_type=jnp.float32) + b1_ref[0])
    o_ref[0, 0] = _pack2(jnp.dot(t, w2_ref[0], preferred_element_type=jnp.float32)
                         + b2_ref[0])


def _edge_filters(ed, w1s, b1s, w2s, b2s):
    grid = (3, _NEB)
    return pl.pallas_call(
        _filters_body,
        grid=grid,
        in_specs=[
            pl.BlockSpec((1, 1, _EBLK), lambda l, j: (j, 0, 0)),
            pl.BlockSpec((1, RBF, HID), lambda l, j: (l, 0, 0)),
            pl.BlockSpec((1, 1, HID), lambda l, j: (l, 0, 0)),
            pl.BlockSpec((1, HID, HID), lambda l, j: (l, 0, 0)),
            pl.BlockSpec((1, 1, HID), lambda l, j: (l, 0, 0)),
        ],
        out_specs=pl.BlockSpec((1, 1, _EBLK, HID // 2), lambda l, j: (l, j, 0, 0)),
        out_shape=jax.ShapeDtypeStruct((3, _NEB, _EBLK, HID // 2), jnp.float32),
    )(ed.reshape(_NEB, 1, _EBLK), w1s.reshape(3, RBF, HID),
      b1s.reshape(3, 1, HID), w2s.reshape(3, HID, HID), b2s.reshape(3, 1, HID))


def _sc_body(h_hbm, src_hbm, dst_hbm, w_hbm, zeros_hbm, out_hbm,
             idx0, idx1, rows0, rows1, w0, w1, dst0, dst1,
             acc_sh, si0, si1, sg0, sg1, sw0, sw1, sd0, sd1):
    cid = lax.axis_index("c")
    sid = lax.axis_index("s")
    wid = sid * NC + cid

    pltpu.sync_copy(zeros_hbm, acc_sh.at[pl.ds(sid * RPS, RPS)])
    plsc.subcore_barrier()
    base = wid * EPW

    def fire_idx(j, idxb, si):
        pltpu.async_copy(src_hbm.at[pl.ds(base + j * CH, CH)], idxb, si)

    def wait_idx(j, idxb, si):
        pltpu.make_async_copy(src_hbm.at[pl.ds(base + j * CH, CH)], idxb, si).wait()

    def fire_main(j, idxb, rows, w, dstb, sg, sw, sd):
        pltpu.async_copy(h_hbm.at[idxb], rows, sg)
        pltpu.async_copy(w_hbm.at[pl.ds(base + j * CH, CH)], w, sw)
        pltpu.async_copy(dst_hbm.at[pl.ds(base + j * CH, CH)], dstb, sd)

    def process(j, idxb, rows, w, dstb, sg, sw, sd):
        pltpu.make_async_copy(h_hbm.at[idxb], rows, sg).wait()
        pltpu.make_async_copy(w_hbm.at[pl.ds(base + j * CH, CH)], w, sw).wait()
        pltpu.make_async_copy(dst_hbm.at[pl.ds(base + j * CH, CH)], dstb, sd).wait()

        def rowblk(r, c2):
            for rr in range(8):
                e = r * 8 + rr
                for cc in range(HID // 32):
                    s = pl.ds(cc * 16, 16)
                    pw = lax.bitcast_convert_type(w[e, s], jnp.int32)
                    wa = lax.bitcast_convert_type(pw << 16, jnp.float32)
                    wb = lax.bitcast_convert_type(pw & jnp.int32(-65536), jnp.float32)
                    rows[e, s] = rows[e, s] * wa
                    s2 = pl.ds(64 + cc * 16, 16)
                    rows[e, s2] = rows[e, s2] * wb
            return c2

        lax.fori_loop(0, CH // 8, rowblk, 0)
        pltpu.sync_copy(rows, acc_sh.at[dstb], add=True)

    b0 = (idx0, rows0, w0, dst0, sg0, sw0, sd0)
    b1 = (idx1, rows1, w1, dst1, sg1, sw1, sd1)

    fire_idx(0, idx0, si0)
    fire_idx(1, idx1, si1)
    wait_idx(0, idx0, si0)
    fire_main(0, *b0)

    def pair(i, carry):
        j0 = i * 2
        j1 = j0 + 1
        wait_idx(j1, idx1, si1)
        fire_main(j1, *b1)

        @pl.when(j0 + 2 < NCH)
        def _():
            fire_idx(j0 + 2, idx0, si0)

        process(j0, *b0)

        @pl.when(j0 + 2 < NCH)
        def _():
            wait_idx(j0 + 2, idx0, si0)
            fire_main(j0 + 2, *b0)

        @pl.when(j1 + 2 < NCH)
        def _():
            fire_idx(j1 + 2, idx1, si1)

        process(j1, *b1)
        return carry

    lax.fori_loop(0, NCH // 2, pair, 0)
    process(NCH - 1, *b0)

    plsc.subcore_barrier()
    pltpu.sync_copy(acc_sh.at[pl.ds(sid * RPS, RPS)],
                    out_hbm.at[cid, pl.ds(sid * RPS, RPS)])


@functools.cache
def _make_sc_agg():
    return pl.kernel(
        _sc_body,
        mesh=plsc.VectorSubcoreMesh(core_axis_name="c", subcore_axis_name="s"),
        out_type=jax.ShapeDtypeStruct((NC, NPAD, HID), jnp.float32),
        scratch_types=[
            pltpu.VMEM((CH,), jnp.int32),
            pltpu.VMEM((CH,), jnp.int32),
            pltpu.VMEM((CH, HID), jnp.float32),
            pltpu.VMEM((CH, HID), jnp.float32),
            pltpu.VMEM((CH, HID // 2), jnp.float32),
            pltpu.VMEM((CH, HID // 2), jnp.float32),
            pltpu.VMEM((CH,), jnp.int32),
            pltpu.VMEM((CH,), jnp.int32),
            pltpu.VMEM_SHARED((NPAD, HID), jnp.float32),
        ] + [pltpu.SemaphoreType.DMA] * 8,
    )


def _sc_agg(h, src, dst, w, zeros):
    return _make_sc_agg()(h, src, dst, w, zeros)


def _hup_body(p_ref, h_ref, w_ref, b_ref, o_ref):
    agg = p_ref[0] + p_ref[1]
    o_ref[...] = h_ref[...] + _silu(
        jnp.dot(agg, w_ref[...], preferred_element_type=jnp.float32) + b_ref[...])


def _h_update(partials, h, lw, lb):
    grid = (5,)
    return pl.pallas_call(
        _hup_body,
        grid=grid,
        in_specs=[
            pl.BlockSpec((NC, 2000, HID), lambda i: (0, i, 0)),

            pl.BlockSpec((2000, HID), lambda i: (i, 0)),
            pl.BlockSpec((HID, HID), lambda i: (0, 0)),
            pl.BlockSpec((1, HID), lambda i: (0, 0)),
        ],
        out_specs=pl.BlockSpec((2000, HID), lambda i: (i, 0)),
        out_shape=jax.ShapeDtypeStruct((N_NODES, HID), jnp.float32),
    )(partials, h, lw, lb)


def _head_body(h_ref, batch_ref, chem_ref, qx_ref, qmask_ref,
               gw1, gb1, gw2r, gb2, pw, pb, pg, pbeta,
               cw1, cb1, cg1, cbeta1, cw2, cb2, cg2, cbeta2,
               qw1, qb1, qw2, qb2, qmiss,
               fvw, fvb, fow, fob, fg, fbeta,
               hw1a, hw1b, hb1, hg1, hbeta1, hw2, hb2, hw3r, hb3,
               o_ref):
    h = h_ref[...]
    gate_h = _silu(jnp.dot(h, gw1[...], preferred_element_type=jnp.float32) + gb1[...])
    gate = jnp.sum(gate_h * gw2r[...], axis=1, keepdims=True) + gb2[...]

    cols = lax.broadcasted_iota(jnp.int32, (N_NODES, B), 1)
    maskf = (batch_ref[...] == cols).astype(jnp.float32)

    gmax_g = jnp.max(jnp.where(maskf > 0.0, gate, -1e30), axis=0, keepdims=True)
    gmax_n = lax.dot_general(maskf, gmax_g, (((1,), (1,)), ((), ())),
                             preferred_element_type=jnp.float32)
    gexp = jnp.exp(gate - gmax_n)
    gsum_g = lax.dot_general(maskf, gexp, (((0,), (0,)), ((), ())),
                             preferred_element_type=jnp.float32)
    gsum_n = lax.dot_general(maskf, gsum_g, (((1,), (0,)), ((), ())),
                             preferred_element_type=jnp.float32) + 1e-8
    alpha = gexp / gsum_n
    hg = lax.dot_general(maskf, alpha * h, (((0,), (0,)), ((), ())),
                         preferred_element_type=jnp.float32)

    g = _gelu(_ln(jnp.dot(hg, pw[...], preferred_element_type=jnp.float32) + pb[...],
                  pg[...], pbeta[...]))

    c = _gelu(_ln(jnp.dot(chem_ref[...], cw1[...], preferred_element_type=jnp.float32)
                  + cb1[...], cg1[...], cbeta1[...]))
    c = _gelu(_ln(jnp.dot(c, cw2[...], preferred_element_type=jnp.float32) + cb2[...],
                  cg2[...], cbeta2[...]))

    qf = _gelu(jnp.dot(qx_ref[...], qw1[...], preferred_element_type=jnp.float32) + qb1[...])
    qf = _gelu(jnp.dot(qf, qw2[...], preferred_element_type=jnp.float32) + qb2[...])
    q_out = jnp.where(qmask_ref[...] > 0, qf, qmiss[...])

    vv = jnp.dot(g, fvw[...], preferred_element_type=jnp.float32) + fvb[...]
    fo = jnp.dot(vv, fow[...], preferred_element_type=jnp.float32) + fob[...]
    fo = _ln(fo + c, fg[...], fbeta[...])

    x = _gelu(_ln(jnp.dot(fo, hw1a[...], preferred_element_type=jnp.float32)
                  + jnp.dot(q_out, hw1b[...], preferred_element_type=jnp.float32)
                  + hb1[...], hg1[...], hbeta1[...]))
    x = _gelu(jnp.dot(x, hw2[...], preferred_element_type=jnp.float32) + hb2[...])
    o_ref[...] = jnp.sum(x * hw3r[...], axis=1, keepdims=True) + hb3[...]


def _head(h, batch2d, chem, qx, qmask2d, weights):
    return pl.pallas_call(
        _head_body,
        out_shape=jax.ShapeDtypeStruct((B, 1), jnp.float32),
    )(h, batch2d, chem, qx, qmask2d, *weights)


def kernel(params, node_features, edge_dist, chemical_x, quantum_x, edge_index,
           batch, quantum_mask):
    p = params
    f32 = jnp.float32

    nf8 = jnp.pad(node_features, ((0, 0), (0, 5)))
    w8 = jnp.pad(p['ne_W'], ((0, 5), (0, 0)))
    h = _node_embed(nf8, w8, p['ne_b'].reshape(1, HID))

    w1s = jnp.stack([p['int%d' % i]['fW1'] for i in range(3)])
    b1s = jnp.stack([p['int%d' % i]['fb1'] for i in range(3)])
    w2s = jnp.stack([p['int%d' % i]['fW2'] for i in range(3)])
    b2s = jnp.stack([p['int%d' % i]['fb2'] for i in range(3)])
    W = _edge_filters(edge_dist, w1s, b1s, w2s, b2s).reshape(3, N_EDGES, HID // 2)

    src = edge_index[0].astype(jnp.int32)
    dst = edge_index[1].astype(jnp.int32)
    zeros = jnp.zeros((RPS, HID), jnp.float32)

    for l in range(3):
        partials = _sc_agg(h, src, dst, W[l], zeros)
        h = _h_update(partials, h, p['int%d' % l]['lW'],
                      p['int%d' % l]['lb'].reshape(1, HID))

    weights = [
        p['gate_W1'], p['gate_b1'].reshape(1, 64),
        p['gate_W2'].reshape(1, 64), p['gate_b2'].reshape(1, 1),
        p['proj_W'], p['proj_b'].reshape(1, HID),
        p['proj_g'].reshape(1, HID), p['proj_beta'].reshape(1, HID),
        p['chem_W1'], p['chem_b1'].reshape(1, 256),
        p['chem_g1'].reshape(1, 256), p['chem_beta1'].reshape(1, 256),
        p['chem_W2'], p['chem_b2'].reshape(1, 128),
        p['chem_g2'].reshape(1, 128), p['chem_beta2'].reshape(1, 128),
        p['qm_W1'], p['qm_b1'].reshape(1, 64),
        p['qm_W2'], p['qm_b2'].reshape(1, 64), p['qm_missing'].reshape(1, 64),
        p['fu_vW'], p['fu_vb'].reshape(1, 128),
        p['fu_oW'], p['fu_ob'].reshape(1, 128),
        p['fu_g'].reshape(1, 128), p['fu_beta'].reshape(1, 128),
        p['hd_W1'][:128], p['hd_W1'][128:],
        p['hd_b1'].reshape(1, 256),
        p['hd_g1'].reshape(1, 256), p['hd_beta1'].reshape(1, 256),
        p['hd_W2'], p['hd_b2'].reshape(1, 128),
        p['hd_W3'].reshape(1, 128), p['hd_b3'].reshape(1, 1),
    ]
    preds = _head(h, batch.reshape(N_NODES, 1).astype(jnp.int32),
                  chemical_x, quantum_x,
                  quantum_mask.reshape(B, 1).astype(jnp.int32), weights)
    return preds.reshape(B)
